# Initial kernel scaffold; baseline (speedup 1.0000x reference)
import jax
import jax.numpy as jnp
from jax.experimental import pallas as pl


def kernel(X, edge_index, W1, b1, Wm, bm, Ws, bs):
    raise NotImplementedError("write your pallas kernel here")


# speedup vs baseline: 12.8180x; 12.8180x over previous
import functools

import jax
import jax.numpy as jnp
from jax import lax
from jax.experimental import pallas as pl
from jax.experimental.pallas import tpu as pltpu
from jax.experimental.pallas import tpu_sc as plsc

N = 10000
NP = 10240
E = 320000
IN_DIM = 256
HID = 128
LAT = 64

NC = 2
NS = 16
CH = 128
CPW = 79
EPAD = NC * NS * CPW * CH
STRIPE = NP // NS

_F32 = jnp.float32
_HI = lax.Precision.HIGHEST


def _mesh():
    return plsc.VectorSubcoreMesh(core_axis_name="c", subcore_axis_name="s")


def _sc_degree_body(dst_hbm, out_hbm, dst_v, ones_v, acc_sh, sem):
    core = lax.axis_index("c")
    sid = lax.axis_index("s")
    w = core * NS + sid

    @pl.loop(0, CH)
    def _fillz(i):
        @pl.loop(0, HID, step=16)
        def _fillz2(j):
            ones_v[i, pl.ds(j, 16)] = jnp.zeros((16,), _F32)

    @pl.loop(0, STRIPE, step=CH)
    def _zero(r):
        pltpu.sync_copy(ones_v, acc_sh.at[pl.ds(sid * STRIPE + r, CH)])

    @pl.loop(0, CH)
    def _fill1(i):
        @pl.loop(0, HID, step=16)
        def _fill12(j):
            ones_v[i, pl.ds(j, 16)] = jnp.ones((16,), _F32)

    plsc.subcore_barrier()
    pltpu.sync_copy(dst_hbm.at[w], dst_v)

    @pl.loop(0, CPW)
    def _scat(c):
        pltpu.sync_copy(ones_v, acc_sh.at[dst_v.at[c]], add=True)

    plsc.subcore_barrier()
    pltpu.sync_copy(
        acc_sh.at[pl.ds(sid * STRIPE, STRIPE)],
        out_hbm.at[core, pl.ds(sid * STRIPE, STRIPE)],
    )


def _sc_degree(dst3):
    kern = pl.kernel(
        _sc_degree_body,
        out_type=jax.ShapeDtypeStruct((NC, NP, HID), _F32),
        mesh=_mesh(),
        scratch_types=[
            pltpu.VMEM((CPW, CH), jnp.int32),
            pltpu.VMEM((CH, HID), _F32),
            pltpu.VMEM_SHARED((NP, HID), _F32),
            pltpu.SemaphoreType.DMA,
        ],
    )
    return kern(dst3)


def _sc_agg_body(h_hbm, src_hbm, dst_hbm, out_hbm, src_v, dst_v, rows_v, acc_sh, sem):
    core = lax.axis_index("c")
    sid = lax.axis_index("s")
    w = core * NS + sid

    @pl.loop(0, CH)
    def _fill(i):
        @pl.loop(0, HID, step=16)
        def _fill2(j):
            rows_v[i, pl.ds(j, 16)] = jnp.zeros((16,), _F32)

    @pl.loop(0, STRIPE, step=CH)
    def _zero(r):
        pltpu.sync_copy(rows_v, acc_sh.at[pl.ds(sid * STRIPE + r, CH)])

    plsc.subcore_barrier()
    pltpu.sync_copy(src_hbm.at[w], src_v)
    pltpu.sync_copy(dst_hbm.at[w], dst_v)

    @pl.loop(0, CPW)
    def _edge(c):
        pltpu.async_copy(h_hbm.at[src_v.at[c]], rows_v, sem).wait()
        pltpu.sync_copy(rows_v, acc_sh.at[dst_v.at[c]], add=True)

    plsc.subcore_barrier()
    pltpu.sync_copy(
        acc_sh.at[pl.ds(sid * STRIPE, STRIPE)],
        out_hbm.at[core, pl.ds(sid * STRIPE, STRIPE)],
    )


def _sc_aggregate(hp, src3, dst3):
    kern = pl.kernel(
        _sc_agg_body,
        out_type=jax.ShapeDtypeStruct((NC, NP, HID), _F32),
        mesh=_mesh(),
        scratch_types=[
            pltpu.VMEM((CPW, CH), jnp.int32),
            pltpu.VMEM((CPW, CH), jnp.int32),
            pltpu.VMEM((CH, HID), _F32),
            pltpu.VMEM_SHARED((NP, HID), _F32),
            pltpu.SemaphoreType.DMA,
        ],
    )
    return kern(hp, src3, dst3)


def _dinv_of(deg_ref):
    return lax.rsqrt(deg_ref[0, :, 0] + deg_ref[1, :, 0] + 1.0)[:, None]


def _tc_h1_body(x_ref, w_ref, deg_ref, o_ref):
    h = jnp.dot(x_ref[...], w_ref[...], precision=_HI, preferred_element_type=_F32)
    o_ref[...] = h * _dinv_of(deg_ref)


def _tc_h1(xp, w1, deg2):
    bm = 1024
    return pl.pallas_call(
        _tc_h1_body,
        grid=(NP // bm,),
        in_specs=[
            pl.BlockSpec((bm, IN_DIM), lambda i: (i, 0)),
            pl.BlockSpec((IN_DIM, HID), lambda i: (0, 0)),
            pl.BlockSpec((NC, bm, HID), lambda i: (0, i, 0)),
        ],
        out_specs=pl.BlockSpec((bm, HID), lambda i: (i, 0)),
        out_shape=jax.ShapeDtypeStruct((NP, HID), _F32),
    )(xp, w1, deg2)


def _tc_h2_body(acc_ref, h1_ref, deg_ref, b1_ref, wc_ref, o_ref):
    dinv = _dinv_of(deg_ref)
    hidden = jnp.maximum(dinv * (acc_ref[0] + acc_ref[1] + h1_ref[...]) + b1_ref[...], 0.0)
    o_ref[...] = jnp.dot(hidden, wc_ref[...], precision=_HI, preferred_element_type=_F32) * dinv


def _tc_h2(acc1, h1p, deg2, b1, wc):
    bm = 1024
    return pl.pallas_call(
        _tc_h2_body,
        grid=(NP // bm,),
        in_specs=[
            pl.BlockSpec((NC, bm, HID), lambda i: (0, i, 0)),
            pl.BlockSpec((bm, HID), lambda i: (i, 0)),
            pl.BlockSpec((NC, bm, HID), lambda i: (0, i, 0)),
            pl.BlockSpec((1, HID), lambda i: (0, 0)),
            pl.BlockSpec((HID, HID), lambda i: (0, 0)),
        ],
        out_specs=pl.BlockSpec((bm, HID), lambda i: (i, 0)),
        out_shape=jax.ShapeDtypeStruct((NP, HID), _F32),
    )(acc1, h1p, deg2, b1, wc)


def _tc_epi_body(acc_ref, h2_ref, deg_ref, bm_ref, bs_ref, noise_ref,
                 mean_ref, logstd_ref, z_ref):
    t = _dinv_of(deg_ref) * (acc_ref[0] + acc_ref[1] + h2_ref[...])
    mean = t[:, :LAT] + bm_ref[...]
    logstd = t[:, LAT:] + bs_ref[...]
    mean_ref[...] = mean
    logstd_ref[...] = logstd
    z_ref[...] = mean + noise_ref[...] * jnp.exp(logstd)


def _tc_epilogue(acc2, h2p, deg2, bmv, bsv, noise_p):
    bm = 1024
    spec_lat = pl.BlockSpec((bm, LAT), lambda i: (i, 0))
    out = jax.ShapeDtypeStruct((NP, LAT), _F32)
    return pl.pallas_call(
        _tc_epi_body,
        grid=(NP // bm,),
        in_specs=[
            pl.BlockSpec((NC, bm, HID), lambda i: (0, i, 0)),
            pl.BlockSpec((bm, HID), lambda i: (i, 0)),
            pl.BlockSpec((NC, bm, HID), lambda i: (0, i, 0)),
            pl.BlockSpec((1, LAT), lambda i: (0, 0)),
            pl.BlockSpec((1, LAT), lambda i: (0, 0)),
            spec_lat,
        ],
        out_specs=(spec_lat, spec_lat, spec_lat),
        out_shape=(out, out, out),
    )(acc2, h2p, deg2, bmv, bsv, noise_p)


def _tc_adj_body(a_ref, b_ref, o_ref):
    logits = lax.dot_general(
        a_ref[...], b_ref[...], (((1,), (1,)), ((), ())),
        precision=_HI, preferred_element_type=_F32,
    )
    o_ref[...] = jax.nn.sigmoid(logits)


def _tc_adj(z):
    bm = 1024
    return pl.pallas_call(
        _tc_adj_body,
        grid=(pl.cdiv(N, bm), pl.cdiv(N, bm)),
        in_specs=[
            pl.BlockSpec((bm, LAT), lambda i, j: (i, 0)),
            pl.BlockSpec((bm, LAT), lambda i, j: (j, 0)),
        ],
        out_specs=pl.BlockSpec((bm, bm), lambda i, j: (i, j)),
        out_shape=jax.ShapeDtypeStruct((N, N), _F32),
    )(z, z)


def kernel(X, edge_index, W1, b1, Wm, bm, Ws, bs):
    pad = (N + (jnp.arange(EPAD - E, dtype=jnp.int32) & 15)).astype(jnp.int32)
    src3 = jnp.concatenate([edge_index[0], pad]).reshape(NC * NS, CPW, CH)
    dst3 = jnp.concatenate([edge_index[1], pad]).reshape(NC * NS, CPW, CH)
    xp = jnp.pad(X, ((0, NP - N), (0, 0)))

    deg2 = _sc_degree(dst3)
    h1p = _tc_h1(xp, W1, deg2)
    acc1 = _sc_aggregate(h1p, src3, dst3)
    wc = jnp.concatenate([Wm, Ws], axis=1)
    h2p = _tc_h2(acc1, h1p, deg2, b1.reshape(1, HID), wc)
    acc2 = _sc_aggregate(h2p, src3, dst3)

    noise = jax.random.normal(jax.random.key(42), (N, LAT), dtype=_F32)
    noise_p = jnp.pad(noise, ((0, NP - N), (0, 0)))
    mean_p, logstd_p, z_p = _tc_epilogue(
        acc2, h2p, deg2, bm.reshape(1, LAT), bs.reshape(1, LAT), noise_p)
    mean, logstd, z = mean_p[:N], logstd_p[:N], z_p[:N]
    adj = _tc_adj(z)
    return (adj, mean, logstd, z)


# speedup vs baseline: 14.1415x; 1.1033x over previous
import functools

import jax
import jax.numpy as jnp
from jax import lax
from jax.experimental import pallas as pl
from jax.experimental.pallas import tpu as pltpu
from jax.experimental.pallas import tpu_sc as plsc

N = 10000
NP = 10240
E = 320000
IN_DIM = 256
HID = 128
LAT = 64

NC = 2
NS = 16
CH = 128
CPW = 80
HALF = 40
EPAD = NC * NS * CPW * CH
STRIPE = NP // NS

_F32 = jnp.float32
_HI = lax.Precision.HIGHEST


def _mesh():
    return plsc.VectorSubcoreMesh(core_axis_name="c", subcore_axis_name="s")


def _sc_degree_body(dst_hbm, out_hbm, dst_v, ones_v, acc_sh, sem):
    core = lax.axis_index("c")
    sid = lax.axis_index("s")
    w = core * NS + sid

    @pl.loop(0, CH)
    def _fillz(i):
        @pl.loop(0, HID, step=16)
        def _fillz2(j):
            ones_v[i, pl.ds(j, 16)] = jnp.zeros((16,), _F32)

    @pl.loop(0, STRIPE, step=CH)
    def _zero(r):
        pltpu.sync_copy(ones_v, acc_sh.at[pl.ds(sid * STRIPE + r, CH)])

    @pl.loop(0, CH)
    def _fill1(i):
        @pl.loop(0, HID, step=16)
        def _fill12(j):
            ones_v[i, pl.ds(j, 16)] = jnp.ones((16,), _F32)

    plsc.subcore_barrier()
    pltpu.sync_copy(dst_hbm.at[w], dst_v)

    @pl.loop(0, CPW)
    def _scat(c):
        pltpu.sync_copy(ones_v, acc_sh.at[dst_v.at[c]], add=True)

    plsc.subcore_barrier()
    pltpu.sync_copy(
        acc_sh.at[pl.ds(sid * STRIPE, STRIPE)],
        out_hbm.at[core, pl.ds(sid * STRIPE, STRIPE)],
    )


def _sc_degree(dst3):
    kern = pl.kernel(
        _sc_degree_body,
        out_type=jax.ShapeDtypeStruct((NC, NP, HID), _F32),
        mesh=_mesh(),
        scratch_types=[
            pltpu.VMEM((CPW, CH), jnp.int32),
            pltpu.VMEM((CH, HID), _F32),
            pltpu.VMEM_SHARED((NP, HID), _F32),
            pltpu.SemaphoreType.DMA,
        ],
    )
    return kern(dst3)


def _sc_agg_body(h_hbm, src_hbm, dst_hbm, out_hbm, src_v, dst_v, rows_a, rows_b,
                 acc_sh, sem_a, sem_b):
    core = lax.axis_index("c")
    sid = lax.axis_index("s")
    w = core * NS + sid

    @pl.loop(0, CH)
    def _fill(i):
        @pl.loop(0, HID, step=16)
        def _fill2(j):
            rows_a[i, pl.ds(j, 16)] = jnp.zeros((16,), _F32)

    @pl.loop(0, STRIPE, step=CH)
    def _zero(r):
        pltpu.sync_copy(rows_a, acc_sh.at[pl.ds(sid * STRIPE + r, CH)])

    plsc.subcore_barrier()
    pltpu.sync_copy(dst_hbm.at[w], dst_v)

    for ph in range(CPW // HALF):
        base = ph * HALF
        pltpu.sync_copy(src_hbm.at[w, pl.ds(base, HALF)], src_v)
        pltpu.async_copy(h_hbm.at[src_v.at[0]], rows_a, sem_a)

        @pl.loop(0, HALF, step=2)
        def _edge(c, base=base):
            pltpu.make_async_copy(h_hbm.at[src_v.at[c]], rows_a, sem_a).wait()
            pltpu.async_copy(h_hbm.at[src_v.at[c + 1]], rows_b, sem_b)
            pltpu.sync_copy(rows_a, acc_sh.at[dst_v.at[base + c]], add=True)

            pltpu.make_async_copy(h_hbm.at[src_v.at[c + 1]], rows_b, sem_b).wait()

            @pl.when(c + 2 < HALF)
            def _next():
                pltpu.async_copy(h_hbm.at[src_v.at[c + 2]], rows_a, sem_a)

            pltpu.sync_copy(rows_b, acc_sh.at[dst_v.at[base + c + 1]], add=True)

    plsc.subcore_barrier()
    pltpu.sync_copy(
        acc_sh.at[pl.ds(sid * STRIPE, STRIPE)],
        out_hbm.at[core, pl.ds(sid * STRIPE, STRIPE)],
    )


def _sc_aggregate(hp, src3, dst3):
    kern = pl.kernel(
        _sc_agg_body,
        out_type=jax.ShapeDtypeStruct((NC, NP, HID), _F32),
        mesh=_mesh(),
        scratch_types=[
            pltpu.VMEM((HALF, CH), jnp.int32),
            pltpu.VMEM((CPW, CH), jnp.int32),
            pltpu.VMEM((CH, HID), _F32),
            pltpu.VMEM((CH, HID), _F32),
            pltpu.VMEM_SHARED((NP, HID), _F32),
            pltpu.SemaphoreType.DMA,
            pltpu.SemaphoreType.DMA,
        ],
    )
    return kern(hp, src3, dst3)


def _dinv_of(deg_ref):
    return lax.rsqrt(deg_ref[0, :, 0] + deg_ref[1, :, 0] + 1.0)[:, None]


def _tc_h1_body(x_ref, w_ref, deg_ref, o_ref):
    h = jnp.dot(x_ref[...], w_ref[...], precision=_HI, preferred_element_type=_F32)
    o_ref[...] = h * _dinv_of(deg_ref)


def _tc_h1(xp, w1, deg2):
    bm = 1024
    return pl.pallas_call(
        _tc_h1_body,
        grid=(NP // bm,),
        in_specs=[
            pl.BlockSpec((bm, IN_DIM), lambda i: (i, 0)),
            pl.BlockSpec((IN_DIM, HID), lambda i: (0, 0)),
            pl.BlockSpec((NC, bm, HID), lambda i: (0, i, 0)),
        ],
        out_specs=pl.BlockSpec((bm, HID), lambda i: (i, 0)),
        out_shape=jax.ShapeDtypeStruct((NP, HID), _F32),
    )(xp, w1, deg2)


def _tc_h2_body(acc_ref, h1_ref, deg_ref, b1_ref, wc_ref, o_ref):
    dinv = _dinv_of(deg_ref)
    hidden = jnp.maximum(dinv * (acc_ref[0] + acc_ref[1] + h1_ref[...]) + b1_ref[...], 0.0)
    o_ref[...] = jnp.dot(hidden, wc_ref[...], precision=_HI, preferred_element_type=_F32) * dinv


def _tc_h2(acc1, h1p, deg2, b1, wc):
    bm = 1024
    return pl.pallas_call(
        _tc_h2_body,
        grid=(NP // bm,),
        in_specs=[
            pl.BlockSpec((NC, bm, HID), lambda i: (0, i, 0)),
            pl.BlockSpec((bm, HID), lambda i: (i, 0)),
            pl.BlockSpec((NC, bm, HID), lambda i: (0, i, 0)),
            pl.BlockSpec((1, HID), lambda i: (0, 0)),
            pl.BlockSpec((HID, HID), lambda i: (0, 0)),
        ],
        out_specs=pl.BlockSpec((bm, HID), lambda i: (i, 0)),
        out_shape=jax.ShapeDtypeStruct((NP, HID), _F32),
    )(acc1, h1p, deg2, b1, wc)


def _tc_epi_body(acc_ref, h2_ref, deg_ref, bm_ref, bs_ref, noise_ref,
                 mean_ref, logstd_ref, z_ref):
    t = _dinv_of(deg_ref) * (acc_ref[0] + acc_ref[1] + h2_ref[...])
    mean = t[:, :LAT] + bm_ref[...]
    logstd = t[:, LAT:] + bs_ref[...]
    mean_ref[...] = mean
    logstd_ref[...] = logstd
    z_ref[...] = mean + noise_ref[...] * jnp.exp(logstd)


def _tc_epilogue(acc2, h2p, deg2, bmv, bsv, noise_p):
    bm = 1024
    spec_lat = pl.BlockSpec((bm, LAT), lambda i: (i, 0))
    out = jax.ShapeDtypeStruct((NP, LAT), _F32)
    return pl.pallas_call(
        _tc_epi_body,
        grid=(NP // bm,),
        in_specs=[
            pl.BlockSpec((NC, bm, HID), lambda i: (0, i, 0)),
            pl.BlockSpec((bm, HID), lambda i: (i, 0)),
            pl.BlockSpec((NC, bm, HID), lambda i: (0, i, 0)),
            pl.BlockSpec((1, LAT), lambda i: (0, 0)),
            pl.BlockSpec((1, LAT), lambda i: (0, 0)),
            spec_lat,
        ],
        out_specs=(spec_lat, spec_lat, spec_lat),
        out_shape=(out, out, out),
    )(acc2, h2p, deg2, bmv, bsv, noise_p)


def _tc_adj_body(a_ref, b_ref, o_ref):
    logits = lax.dot_general(
        a_ref[...], b_ref[...], (((1,), (1,)), ((), ())),
        precision=_HI, preferred_element_type=_F32,
    )
    o_ref[...] = jax.nn.sigmoid(logits)


def _tc_adj(z):
    bm = 1024
    return pl.pallas_call(
        _tc_adj_body,
        grid=(pl.cdiv(N, bm), pl.cdiv(N, bm)),
        in_specs=[
            pl.BlockSpec((bm, LAT), lambda i, j: (i, 0)),
            pl.BlockSpec((bm, LAT), lambda i, j: (j, 0)),
        ],
        out_specs=pl.BlockSpec((bm, bm), lambda i, j: (i, j)),
        out_shape=jax.ShapeDtypeStruct((N, N), _F32),
    )(z, z)


def kernel(X, edge_index, W1, b1, Wm, bm, Ws, bs):
    pad = (N + (jnp.arange(EPAD - E, dtype=jnp.int32) & 15)).astype(jnp.int32)
    src3 = jnp.concatenate([edge_index[0], pad]).reshape(NC * NS, CPW, CH)
    dst3 = jnp.concatenate([edge_index[1], pad]).reshape(NC * NS, CPW, CH)
    xp = jnp.pad(X, ((0, NP - N), (0, 0)))

    deg2 = _sc_degree(dst3)
    h1p = _tc_h1(xp, W1, deg2)
    acc1 = _sc_aggregate(h1p, src3, dst3)
    wc = jnp.concatenate([Wm, Ws], axis=1)
    h2p = _tc_h2(acc1, h1p, deg2, b1.reshape(1, HID), wc)
    acc2 = _sc_aggregate(h2p, src3, dst3)

    noise = jax.random.normal(jax.random.key(42), (N, LAT), dtype=_F32)
    noise_p = jnp.pad(noise, ((0, NP - N), (0, 0)))
    mean_p, logstd_p, z_p = _tc_epilogue(
        acc2, h2p, deg2, bm.reshape(1, LAT), bs.reshape(1, LAT), noise_p)
    mean, logstd, z = mean_p[:N], logstd_p[:N], z_p[:N]
    adj = _tc_adj(z)
    return (adj, mean, logstd, z)
